# baseline (speedup 1.0000x reference)
import functools

import jax
import jax.numpy as jnp
from jax import lax
from jax.experimental import pallas as pl
from jax.experimental.pallas import tpu as pltpu
from jax.experimental.pallas import tpu_sc as plsc

_NUM_EMBEDDINGS = 1000000
_DIM = 64
_BATCH = 16384
_HIST = 50

_NC = 2
_NS = 16
_NW = _NC * _NS
_ROWS_W = _BATCH // _NW
_BB = 128
_NBB = _ROWS_W // _BB
_NBLK = _HIST * _NBB
_T = _NBLK // 2


@functools.partial(
    pl.kernel,
    out_type=jax.ShapeDtypeStruct(
        (_HIST, _DIM // 8, _BATCH // 128, 8, 128), jnp.float32
    ),
    mesh=plsc.VectorSubcoreMesh(core_axis_name="c", subcore_axis_name="s"),
    scratch_types=[
        pltpu.VMEM((_ROWS_W, _HIST), jnp.int32),
        pltpu.VMEM((_HIST, _ROWS_W), jnp.int32),
        pltpu.VMEM((2, _BB, _DIM), jnp.float32),
        pltpu.VMEM((2, _DIM // 8, 1, 8, 128), jnp.float32),
        pltpu.SemaphoreType.DMA,
        pltpu.SemaphoreType.DMA,
    ],
    compiler_params=pltpu.CompilerParams(
        use_tc_tiling_on_sc=False,
        needs_layout_passes=False,
        disable_bounds_checks=True,
    ),
)
def _sc_gather(table_hbm, idx_hbm, out_hbm, idx_v, idxt_v, rows_v, tiles_v,
               gsem, osem):
    wid = lax.axis_index("s") * _NC + lax.axis_index("c")
    base = wid * _ROWS_W
    pltpu.sync_copy(idx_hbm.at[pl.ds(base, _ROWS_W)], idx_v)

    iota = lax.iota(jnp.int32, 16)
    lane_base = [iota + 16 * j for j in range(_ROWS_W // 16)]

    def idxt_body(h, carry):
        col = jnp.broadcast_to(h, (16,)).astype(jnp.int32)
        for k in range(_ROWS_W // 16):
            v = plsc.load_gather(idx_v, [lane_base[k], col])
            idxt_v[h, pl.ds(16 * k, 16)] = v
        return carry

    lax.fori_loop(0, _HIST, idxt_body, 0)

    def fire_gather(i, s):
        h = i // _NBB
        bb = i - h * _NBB
        pltpu.make_async_copy(
            table_hbm.at[idxt_v.at[h, pl.ds(bb * _BB, _BB)]],
            rows_v.at[s],
            gsem,
        ).start()

    def wait_gather(i, s):
        h = i // _NBB
        bb = i - h * _NBB
        pltpu.make_async_copy(
            table_hbm.at[idxt_v.at[h, pl.ds(bb * _BB, _BB)]],
            rows_v.at[s],
            gsem,
        ).wait()

    def transpose_block(s):
        for dblk in range(_DIM // 8):
            for r in range(8):
                col = jnp.broadcast_to(jnp.int32(8 * dblk + r), (16,))
                for j in range(_BB // 16):
                    v = plsc.load_gather(rows_v.at[s], [lane_base[j], col])
                    tiles_v[s, dblk, 0, r, pl.ds(16 * j, 16)] = v

    def out_copy(i, s, fire):
        h = i // _NBB
        bb = i - h * _NBB
        cp = pltpu.make_async_copy(
            tiles_v.at[s],
            out_hbm.at[h, :, pl.ds(wid * _NBB + bb, 1)],
            osem,
        )
        cp.start() if fire else cp.wait()

    fire_gather(0, 0)

    def body(t, carry):
        i0 = 2 * t
        i1 = i0 + 1
        wait_gather(i0, 0)
        fire_gather(i1, 1)

        @pl.when(t > 0)
        def _():
            out_copy(i0 - 2, 0, False)

        transpose_block(0)
        out_copy(i0, 0, True)

        wait_gather(i1, 1)

        @pl.when(t < _T - 1)
        def _():
            fire_gather(i0 + 2, 0)

        @pl.when(t > 0)
        def _():
            out_copy(i1 - 2, 1, False)

        transpose_block(1)
        out_copy(i1, 1, True)
        return carry

    lax.fori_loop(0, _T, body, 0)
    out_copy(_NBLK - 2, 0, False)
    out_copy(_NBLK - 1, 1, False)


def kernel(input, weight):
    out5 = _sc_gather(weight, input.astype(jnp.int32))
    return jnp.transpose(out5, (2, 4, 0, 1, 3)).reshape(_BATCH, _HIST, _DIM)


# speedup vs baseline: 1.5446x; 1.5446x over previous
import functools

import jax
import jax.numpy as jnp
from jax import lax
from jax.experimental import pallas as pl
from jax.experimental.pallas import tpu as pltpu
from jax.experimental.pallas import tpu_sc as plsc

_NUM_EMBEDDINGS = 1000000
_DIM = 64
_BATCH = 16384
_HIST = 50

_NC = 2
_NS = 16
_NW = _NC * _NS
_ROWS_W = _BATCH // _NW
_BB = 128
_NBB = _ROWS_W // _BB
_NBLK = _HIST * _NBB
_T = _NBLK // 2


@functools.partial(
    pl.kernel,
    out_type=jax.ShapeDtypeStruct(
        (_HIST, _DIM // 8, _BATCH // 128, 8, 128), jnp.float32
    ),
    mesh=plsc.VectorSubcoreMesh(core_axis_name="c", subcore_axis_name="s"),
    scratch_types=[
        pltpu.VMEM((_ROWS_W, _HIST), jnp.int32),
        pltpu.VMEM((_HIST, _ROWS_W), jnp.int32),
        pltpu.VMEM((2, _BB, _DIM), jnp.float32),
        pltpu.VMEM((2, _DIM // 8, 1, 8, 128), jnp.float32),
        pltpu.SemaphoreType.DMA,
        pltpu.SemaphoreType.DMA,
    ],
    compiler_params=pltpu.CompilerParams(
        use_tc_tiling_on_sc=False,
        needs_layout_passes=False,
        disable_bounds_checks=True,
    ),
)
def _sc_gather(table_hbm, idx_hbm, out_hbm, idx_v, idxt_v, rows_v, tiles_v,
               gsem, osem):
    wid = lax.axis_index("s") * _NC + lax.axis_index("c")
    base = wid * _ROWS_W
    pltpu.sync_copy(idx_hbm.at[pl.ds(base, _ROWS_W)], idx_v)

    iota = lax.iota(jnp.int32, 16)
    lane_base = [iota + 16 * j for j in range(_ROWS_W // 16)]

    def idxt_body(h, carry):
        col = jnp.broadcast_to(h, (16,)).astype(jnp.int32)
        for k in range(_ROWS_W // 16):
            v = plsc.load_gather(idx_v, [lane_base[k], col])
            idxt_v[h, pl.ds(16 * k, 16)] = v
        return carry

    lax.fori_loop(0, _HIST, idxt_body, 0)

    def fire_gather(i, s):
        h = i // _NBB
        bb = i - h * _NBB
        pltpu.make_async_copy(
            table_hbm.at[idxt_v.at[h, pl.ds(bb * _BB, _BB)]],
            rows_v.at[s],
            gsem,
        ).start()

    def wait_gather(i, s):
        h = i // _NBB
        bb = i - h * _NBB
        pltpu.make_async_copy(
            table_hbm.at[idxt_v.at[h, pl.ds(bb * _BB, _BB)]],
            rows_v.at[s],
            gsem,
        ).wait()

    def transpose_block(s):
        @plsc.parallel_loop(0, _DIM, unroll=8)
        def _(d):
            dblk = d // 8
            r = d - dblk * 8
            col = jnp.broadcast_to(d, (16,)).astype(jnp.int32)
            for j in range(_BB // 16):
                v = plsc.load_gather(rows_v.at[s], [lane_base[j], col])
                tiles_v[s, dblk, 0, r, pl.ds(16 * j, 16)] = v

    def out_copy(i, s, fire):
        h = i // _NBB
        bb = i - h * _NBB
        cp = pltpu.make_async_copy(
            tiles_v.at[s],
            out_hbm.at[h, :, pl.ds(wid * _NBB + bb, 1)],
            osem,
        )
        cp.start() if fire else cp.wait()

    fire_gather(0, 0)

    def body(t, carry):
        i0 = 2 * t
        i1 = i0 + 1
        wait_gather(i0, 0)
        fire_gather(i1, 1)

        @pl.when(t > 0)
        def _():
            out_copy(i0 - 2, 0, False)

        transpose_block(0)
        out_copy(i0, 0, True)

        wait_gather(i1, 1)

        @pl.when(t < _T - 1)
        def _():
            fire_gather(i0 + 2, 0)

        @pl.when(t > 0)
        def _():
            out_copy(i1 - 2, 1, False)

        transpose_block(1)
        out_copy(i1, 1, True)
        return carry

    lax.fori_loop(0, _T, body, 0)
    out_copy(_NBLK - 2, 0, False)
    out_copy(_NBLK - 1, 1, False)


def kernel(input, weight):
    out5 = _sc_gather(weight, input.astype(jnp.int32))
    return jnp.transpose(out5, (2, 4, 0, 1, 3)).reshape(_BATCH, _HIST, _DIM)


# speedup vs baseline: 1.5764x; 1.0206x over previous
import functools

import jax
import jax.numpy as jnp
from jax import lax
from jax.experimental import pallas as pl
from jax.experimental.pallas import tpu as pltpu
from jax.experimental.pallas import tpu_sc as plsc

_NUM_EMBEDDINGS = 1000000
_DIM = 64
_BATCH = 16384
_HIST = 50

_NC = 2
_NS = 16
_NW = _NC * _NS
_ROWS_W = _BATCH // _NW
_BB = 128
_NBB = _ROWS_W // _BB
_NBLK = _HIST * _NBB
_T = _NBLK // 2


@functools.partial(
    pl.kernel,
    out_type=jax.ShapeDtypeStruct(
        (_HIST, _DIM // 8, _BATCH // 128, 8, 128), jnp.float32
    ),
    mesh=plsc.VectorSubcoreMesh(core_axis_name="c", subcore_axis_name="s"),
    scratch_types=[
        pltpu.VMEM((_ROWS_W, _HIST), jnp.int32),
        pltpu.VMEM((_HIST, _ROWS_W), jnp.int32),
        pltpu.VMEM((2, _BB, _DIM), jnp.float32),
        pltpu.VMEM((2, _DIM // 8, 1, 8, 128), jnp.float32),
        pltpu.SemaphoreType.DMA,
        pltpu.SemaphoreType.DMA,
    ],
    compiler_params=pltpu.CompilerParams(
        use_tc_tiling_on_sc=False,
        needs_layout_passes=False,
        disable_bounds_checks=True,
    ),
)
def _sc_gather(table_hbm, idx_hbm, out_hbm, idx_v, idxt_v, rows_v, tiles_v,
               gsem, osem):
    wid = lax.axis_index("s") * _NC + lax.axis_index("c")
    base = wid * _ROWS_W
    pltpu.sync_copy(idx_hbm.at[pl.ds(base, _ROWS_W)], idx_v)

    iota = lax.iota(jnp.int32, 16)
    lane_base = [iota + 16 * j for j in range(_ROWS_W // 16)]

    def idxt_body(h, carry):
        col = jnp.broadcast_to(h, (16,)).astype(jnp.int32)
        for k in range(_ROWS_W // 16):
            v = plsc.load_gather(idx_v, [lane_base[k], col])
            idxt_v[h, pl.ds(16 * k, 16)] = v
        return carry

    lax.fori_loop(0, _HIST, idxt_body, 0)

    def fire_gather(i, s):
        h = i // _NBB
        bb = i - h * _NBB
        pltpu.make_async_copy(
            table_hbm.at[idxt_v.at[h, pl.ds(bb * _BB, _BB)]],
            rows_v.at[s],
            gsem,
        ).start()

    def wait_gather(i, s):
        h = i // _NBB
        bb = i - h * _NBB
        pltpu.make_async_copy(
            table_hbm.at[idxt_v.at[h, pl.ds(bb * _BB, _BB)]],
            rows_v.at[s],
            gsem,
        ).wait()

    def transpose_block(s):
        @plsc.parallel_loop(0, _DIM, unroll=16)
        def _(d):
            dblk = d // 8
            r = d - dblk * 8
            col = jnp.broadcast_to(d, (16,)).astype(jnp.int32)
            for j in range(_BB // 16):
                v = plsc.load_gather(rows_v.at[s], [lane_base[j], col])
                tiles_v[s, dblk, 0, r, pl.ds(16 * j, 16)] = v

    def out_copy(i, s, fire):
        h = i // _NBB
        bb = i - h * _NBB
        cp = pltpu.make_async_copy(
            tiles_v.at[s],
            out_hbm.at[h, :, pl.ds(wid * _NBB + bb, 1)],
            osem,
        )
        cp.start() if fire else cp.wait()

    fire_gather(0, 0)

    def body(t, carry):
        i0 = 2 * t
        i1 = i0 + 1
        wait_gather(i0, 0)
        fire_gather(i1, 1)

        @pl.when(t > 0)
        def _():
            out_copy(i0 - 2, 0, False)

        transpose_block(0)
        out_copy(i0, 0, True)

        wait_gather(i1, 1)

        @pl.when(t < _T - 1)
        def _():
            fire_gather(i0 + 2, 0)

        @pl.when(t > 0)
        def _():
            out_copy(i1 - 2, 1, False)

        transpose_block(1)
        out_copy(i1, 1, True)
        return carry

    lax.fori_loop(0, _T, body, 0)
    out_copy(_NBLK - 2, 0, False)
    out_copy(_NBLK - 1, 1, False)


def kernel(input, weight):
    out5 = _sc_gather(weight, input.astype(jnp.int32))
    return jnp.transpose(out5, (2, 4, 0, 1, 3)).reshape(_BATCH, _HIST, _DIM)
